# Initial kernel scaffold; baseline (speedup 1.0000x reference)
import jax
import jax.numpy as jnp
from jax.experimental import pallas as pl


def kernel(x, freqs, Wq, Wdkv, Wuk, Wuv, Wo, g1, b1, g2, b2, Wg, W1, W2):
    raise NotImplementedError("write your pallas kernel here")


# speedup vs baseline: 1.3958x; 1.3958x over previous
import functools

import jax
import jax.numpy as jnp
from jax import lax
from jax.experimental import pallas as pl
from jax.experimental.pallas import tpu as pltpu

_H = 12
_DH = 64


def _preattn_body(x_ref, f_ref, wq_ref, wdkv_ref, wuk_ref, wuv_ref, g1_ref,
                  b1_ref, q_ref, k_ref, v_ref):
    x = x_ref[...]
    m = jnp.mean(x, axis=-1, keepdims=True)
    var = jnp.mean((x - m) ** 2, axis=-1, keepdims=True)
    xn = (x - m) * lax.rsqrt(var + 1e-5) * g1_ref[...] + b1_ref[...]
    q = jnp.dot(xn, wq_ref[...], preferred_element_type=jnp.float32)
    latv = jnp.dot(xn, wdkv_ref[...], preferred_element_type=jnp.float32)
    k = jnp.dot(latv, wuk_ref[...], preferred_element_type=jnp.float32)
    v = jnp.dot(latv, wuv_ref[...], preferred_element_type=jnp.float32)
    f = f_ref[...]
    cos = jnp.cos(f)
    sin = jnp.sin(f)
    cos_t = jnp.concatenate([cos] * _H, axis=1)
    sin_t = jnp.concatenate([sin] * _H, axis=1)

    def rot_half(t):
        parts = []
        for h in range(_H):
            a = t[:, h * _DH:h * _DH + _DH // 2]
            b = t[:, h * _DH + _DH // 2:(h + 1) * _DH]
            parts.append(-b)
            parts.append(a)
        return jnp.concatenate(parts, axis=1)

    q_ref[...] = q * cos_t + rot_half(q) * sin_t
    k_ref[...] = k * cos_t + rot_half(k) * sin_t
    v_ref[...] = v


def _preattn(x, freqs, Wq, Wdkv, Wuk, Wuv, g1, b1):
    S, D = x.shape
    BS = 256
    L = Wdkv.shape[1]
    grid = (S // BS,)
    full = lambda shape: pl.BlockSpec(shape, lambda i: (0,) * len(shape))
    return pl.pallas_call(
        _preattn_body,
        grid=grid,
        in_specs=[
            pl.BlockSpec((BS, D), lambda i: (i, 0)),
            pl.BlockSpec((BS, _DH), lambda i: (i, 0)),
            full((D, D)),
            full((D, L)),
            full((L, D)),
            full((L, D)),
            full((1, D)),
            full((1, D)),
        ],
        out_specs=[
            pl.BlockSpec((BS, D), lambda i: (i, 0)),
            pl.BlockSpec((BS, D), lambda i: (i, 0)),
            pl.BlockSpec((BS, D), lambda i: (i, 0)),
        ],
        out_shape=[jax.ShapeDtypeStruct((S, D), jnp.float32)] * 3,
    )(x, freqs, Wq, Wdkv, Wuk, Wuv, g1.reshape(1, D), b1.reshape(1, D))


def _flash_body(q_ref, k_ref, v_ref, o_ref, *, BQ, BK):
    qi = pl.program_id(1)
    q = q_ref[...] * (1.0 / 8.0)
    qa, qb = q[:, :_DH], q[:, _DH:]
    rows = qi * BQ + lax.broadcasted_iota(jnp.int32, (BQ, 1), 0)

    def body(j, carry):
        acca, ma, la, accb, mb, lb = carry
        kblk = k_ref[pl.ds(j * BK, BK), :]
        vblk = v_ref[pl.ds(j * BK, BK), :]
        cols = j * BK + lax.broadcasted_iota(jnp.int32, (1, BK), 1)
        cmask = cols <= rows

        def one(qh, kh, vh, acc, m, l):
            s = lax.dot_general(qh, kh, (((1,), (1,)), ((), ())),
                                preferred_element_type=jnp.float32)
            s = jnp.where(cmask, s, -1e30)
            m_new = jnp.maximum(m, jnp.max(s, axis=-1, keepdims=True))
            p = jnp.exp(s - m_new)
            alpha = jnp.exp(m - m_new)
            l = l * alpha + jnp.sum(p, axis=-1, keepdims=True)
            acc = acc * alpha + jnp.dot(p, vh,
                                        preferred_element_type=jnp.float32)
            return acc, m_new, l

        acca, ma, la = one(qa, kblk[:, :_DH], vblk[:, :_DH], acca, ma, la)
        accb, mb, lb = one(qb, kblk[:, _DH:], vblk[:, _DH:], accb, mb, lb)
        return acca, ma, la, accb, mb, lb

    acc0 = jnp.zeros((BQ, _DH), jnp.float32)
    m0 = jnp.full((BQ, 1), -jnp.inf, jnp.float32)
    l0 = jnp.zeros((BQ, 1), jnp.float32)
    acca, ma, la, accb, mb, lb = lax.fori_loop(
        0, qi + 1, body, (acc0, m0, l0, acc0, m0, l0))
    o_ref[...] = jnp.concatenate([acca / la, accb / lb], axis=1)


def _flash(q, k, v):
    S, D = q.shape
    BQ = BK = 256
    BH = 2 * _DH
    grid = (_H // 2, S // BQ)
    return pl.pallas_call(
        functools.partial(_flash_body, BQ=BQ, BK=BK),
        grid=grid,
        in_specs=[
            pl.BlockSpec((BQ, BH), lambda h, i: (i, h)),
            pl.BlockSpec((S, BH), lambda h, i: (0, h)),
            pl.BlockSpec((S, BH), lambda h, i: (0, h)),
        ],
        out_specs=pl.BlockSpec((BQ, BH), lambda h, i: (i, h)),
        out_shape=jax.ShapeDtypeStruct((S, D), jnp.float32),
    )(q, k, v)


def _postattn_body(o_ref, x_ref, wo_ref, g2_ref, b2_ref, wg_ref, h_ref,
                   hn_ref, gates_ref, *, E):
    attn = jnp.dot(o_ref[...], wo_ref[...], preferred_element_type=jnp.float32)
    h = x_ref[...] + attn
    h_ref[...] = h
    m = jnp.mean(h, axis=-1, keepdims=True)
    var = jnp.mean((h - m) ** 2, axis=-1, keepdims=True)
    hn = (h - m) * lax.rsqrt(var + 1e-5) * g2_ref[...] + b2_ref[...]
    hn_ref[...] = hn
    logits = jnp.dot(hn, wg_ref[...], preferred_element_type=jnp.float32)
    mx = jnp.max(logits, axis=-1, keepdims=True)
    ex = jnp.exp(logits - mx)
    probs = ex / jnp.sum(ex, axis=-1, keepdims=True)
    S = probs.shape[0]
    ids = lax.broadcasted_iota(jnp.int32, (S, E), 1)
    m1 = jnp.max(probs, axis=-1, keepdims=True)
    i1 = jnp.min(jnp.where(probs == m1, ids, E), axis=-1, keepdims=True)
    p2 = jnp.where(ids == i1, -1.0, probs)
    m2 = jnp.max(p2, axis=-1, keepdims=True)
    i2 = jnp.min(jnp.where(p2 == m2, ids, E), axis=-1, keepdims=True)
    den = m1 + m2
    gates_ref[...] = jnp.where(ids == i1, m1 / den, 0.0) + jnp.where(
        ids == i2, m2 / den, 0.0)


def _postattn(o, x, Wo, g2, b2, Wg):
    S, D = x.shape
    E = Wg.shape[1]
    full = lambda shape: pl.BlockSpec(shape, lambda: (0,) * len(shape))
    return pl.pallas_call(
        functools.partial(_postattn_body, E=E),
        in_specs=[full((S, D)), full((S, D)), full((D, D)), full((1, D)),
                  full((1, D)), full((D, E))],
        out_specs=[full((S, D)), full((S, D)), full((S, E))],
        out_shape=[
            jax.ShapeDtypeStruct((S, D), jnp.float32),
            jax.ShapeDtypeStruct((S, D), jnp.float32),
            jax.ShapeDtypeStruct((S, E), jnp.float32),
        ],
    )(o, x, Wo, g2.reshape(1, D), b2.reshape(1, D), Wg)


def _moe_body(hn_ref, h_ref, g_ref, w1_ref, w2_ref, out_ref):
    e = pl.program_id(0)

    @pl.when(e == 0)
    def _():
        out_ref[...] = h_ref[...]

    gates = g_ref[...]
    ids = lax.broadcasted_iota(jnp.int32, gates.shape, 1)
    gcol = jnp.sum(jnp.where(ids == e, gates, 0.0), axis=1, keepdims=True)
    h1 = jnp.maximum(
        jnp.dot(hn_ref[...], w1_ref[0], preferred_element_type=jnp.float32),
        0.0)
    y = jnp.dot(h1, w2_ref[0], preferred_element_type=jnp.float32)
    out_ref[...] += gcol * y


def _moe(hn, h, gates, W1, W2):
    S, D = hn.shape
    E, _, F = W1.shape
    return pl.pallas_call(
        _moe_body,
        grid=(E,),
        in_specs=[
            pl.BlockSpec((S, D), lambda e: (0, 0)),
            pl.BlockSpec((S, D), lambda e: (0, 0)),
            pl.BlockSpec((S, E), lambda e: (0, 0)),
            pl.BlockSpec((1, D, F), lambda e: (e, 0, 0)),
            pl.BlockSpec((1, F, D), lambda e: (e, 0, 0)),
        ],
        out_specs=pl.BlockSpec((S, D), lambda e: (0, 0)),
        out_shape=jax.ShapeDtypeStruct((S, D), jnp.float32),
    )(hn, h, gates, W1, W2)


@jax.jit
def kernel(x, freqs, Wq, Wdkv, Wuk, Wuv, Wo, g1, b1, g2, b2, Wg, W1, W2):
    B, S, D = x.shape
    x2 = x.reshape(S, D)
    q, k, v = _preattn(x2, freqs, Wq, Wdkv, Wuk, Wuv, g1, b1)
    o = _flash(q, k, v)
    h, hn, gates = _postattn(o, x2, Wo, g2, b2, Wg)
    out = _moe(hn, h, gates, W1, W2)
    k4 = k.reshape(S, _H, _DH).transpose(1, 0, 2).reshape(B, _H, S, _DH)
    v4 = v.reshape(S, _H, _DH).transpose(1, 0, 2).reshape(B, _H, S, _DH)
    return out.reshape(B, S, D), k4, v4
